# baseline (speedup 1.0000x reference)
import functools

import jax
import jax.numpy as jnp
from jax import lax
from jax.experimental import pallas as pl
from jax.experimental.pallas import tpu as pltpu
from jax.experimental.pallas import tpu_sc as plsc

EMB_DIM = 32
LANES = 16


def _make_sc_kernel(batch, emb_dim):
    info = plsc.get_sparse_core_info()
    nc, ns = info.num_cores, info.num_subcores
    nw = nc * ns
    assert batch % (8 * nw) == 0
    b_per_w = batch // nw
    n_groups = b_per_w // LANES
    mesh = plsc.VectorSubcoreMesh(core_axis_name="c", subcore_axis_name="s")

    @functools.partial(
        pl.kernel,
        mesh=mesh,
        out_type=jax.ShapeDtypeStruct((batch,), jnp.float32),
        scratch_types=[
            pltpu.VMEM((b_per_w,), jnp.int32),
            pltpu.VMEM((b_per_w,), jnp.int32),
            pltpu.VMEM((b_per_w, emb_dim), jnp.float32),
            pltpu.VMEM((b_per_w, emb_dim), jnp.float32),
            pltpu.VMEM((b_per_w,), jnp.float32),
            pltpu.VMEM((b_per_w,), jnp.float32),
            pltpu.VMEM((LANES,), jnp.float32),
            pltpu.VMEM((b_per_w,), jnp.float32),
            pltpu.SemaphoreType.DMA,
        ],
        compiler_params=pltpu.CompilerParams(
            needs_layout_passes=False, use_tc_tiling_on_sc=False),
    )
    def k(users_hbm, items_hbm, utab_hbm, itab_hbm, ubias_hbm, ibias_hbm,
          bias_hbm, out_hbm, users_v, items_v, urows_v, irows_v,
          ubias_v, ibias_v, bias_v, out_v, sem):
        wid = lax.axis_index("s") * nc + lax.axis_index("c")
        base = wid * b_per_w

        pltpu.sync_copy(users_hbm.at[pl.ds(base, b_per_w)], users_v)
        pltpu.sync_copy(items_hbm.at[pl.ds(base, b_per_w)], items_v)
        pltpu.sync_copy(bias_hbm, bias_v)

        cp_u = pltpu.make_async_copy(utab_hbm.at[users_v], urows_v, sem)
        cp_i = pltpu.make_async_copy(itab_hbm.at[items_v], irows_v, sem)
        cp_ub = pltpu.make_async_copy(ubias_hbm.at[users_v], ubias_v, sem)
        cp_ib = pltpu.make_async_copy(ibias_hbm.at[items_v], ibias_v, sem)
        cp_u.start()
        cp_i.start()
        cp_ub.start()
        cp_ib.start()
        cp_u.wait()
        cp_i.wait()
        cp_ub.wait()
        cp_ib.wait()

        bias_vec = bias_v[...]
        lane_iota = lax.iota(jnp.int32, LANES)

        def group(g, _):
            row_idx = g * LANES + lane_iota
            acc = jnp.zeros((LANES,), jnp.float32)
            for d in range(emb_dim):
                col = jnp.full((LANES,), d, jnp.int32)
                uv = plsc.load_gather(urows_v, [row_idx, col])
                iv = plsc.load_gather(irows_v, [row_idx, col])
                acc = acc + uv * iv
            sl = pl.ds(g * LANES, LANES)
            acc = acc + ubias_v[sl] + ibias_v[sl] + bias_vec
            out_v[sl] = 1.0 / (1.0 + jnp.exp(-acc))
            return _

        lax.fori_loop(0, n_groups, group, 0)
        pltpu.sync_copy(out_v, out_hbm.at[pl.ds(base, b_per_w)])

    return k


@jax.jit
def kernel(users, items, user_table, item_table, user_bias, item_bias, bias):
    batch = users.shape[0]
    users = users.astype(jnp.int32)
    items = items.astype(jnp.int32)
    bias16 = jnp.broadcast_to(bias.astype(jnp.float32), (LANES,))
    k = _make_sc_kernel(batch, EMB_DIM)
    return k(users, items, user_table, item_table, user_bias, item_bias,
             bias16)
